# Initial kernel scaffold; baseline (speedup 1.0000x reference)
import jax
import jax.numpy as jnp
from jax.experimental import pallas as pl


def kernel(z, edge_index):
    raise NotImplementedError("write your pallas kernel here")


# speedup vs baseline: 1.4148x; 1.4148x over previous
import functools

import jax
import jax.numpy as jnp
from jax import lax
from jax.experimental import pallas as pl
from jax.experimental.pallas import tpu as pltpu
from jax.experimental.pallas import tpu_sc as plsc

_NC = 2
_NS = 16
_NW = _NC * _NS
_L = 16
_D = 128
_C = 80


def _normalize_body(z_ref, out_ref):
    x = z_ref[...]
    ss = jnp.sum(x * x, axis=-1, keepdims=True)
    out_ref[...] = x / jnp.maximum(jnp.sqrt(ss), 1e-12)


def _make_edge_dot(n_edges: int):
    ew = n_edges // _NW
    n_chunks = ew // _C
    n_groups = _C // _L
    mesh = plsc.VectorSubcoreMesh(core_axis_name="c", subcore_axis_name="s")

    @functools.partial(
        pl.kernel,
        mesh=mesh,
        out_type=jax.ShapeDtypeStruct((n_edges,), jnp.float32),
        compiler_params=pltpu.CompilerParams(needs_layout_passes=False),
        scratch_types=[
            pltpu.VMEM((_C,), jnp.int32),
            pltpu.VMEM((_C,), jnp.int32),
            pltpu.VMEM((_C, _D), jnp.float32),
            pltpu.VMEM((_C, _D), jnp.float32),
            pltpu.VMEM((_C,), jnp.float32),
            pltpu.SemaphoreType.DMA,
            pltpu.SemaphoreType.DMA,
        ],
    )
    def edge_dot(zn, u, v, out, idx_u, idx_v, rows_u, rows_v, oc, s1, s2):
        wid = lax.axis_index("s") * _NC + lax.axis_index("c")

        def chunk_body(t, carry):
            base = pl.multiple_of(wid * ew + t * _C, 8)
            pltpu.sync_copy(u.at[pl.ds(base, _C)], idx_u)
            pltpu.sync_copy(v.at[pl.ds(base, _C)], idx_v)
            cu = pltpu.async_copy(zn.at[idx_u], rows_u, s1)
            cv = pltpu.async_copy(zn.at[idx_v], rows_v, s2)
            cu.wait()
            cv.wait()

            def group_body(g, carry2):
                r0 = pl.multiple_of(g * _L, _L)
                rows = r0 + lax.iota(jnp.int32, _L)
                acc = jnp.zeros((_L,), jnp.float32)
                for d in range(_D):
                    col = jnp.full((_L,), d, jnp.int32)
                    a = plsc.load_gather(rows_u, [rows, col])
                    b = plsc.load_gather(rows_v, [rows, col])
                    acc = acc + a * b
                oc[pl.ds(r0, _L)] = acc
                return carry2

            lax.fori_loop(0, n_groups, group_body, 0)
            pltpu.sync_copy(oc, out.at[pl.ds(base, _C)])
            return carry

        lax.fori_loop(0, n_chunks, chunk_body, 0)

    return edge_dot


def kernel(z, edge_index):
    n, d = z.shape
    assert d == _D
    zn = pl.pallas_call(
        _normalize_body,
        out_shape=jax.ShapeDtypeStruct((n, d), jnp.float32),
    )(z)
    u = edge_index[0].astype(jnp.int32)
    v = edge_index[1].astype(jnp.int32)
    n_edges = u.shape[0]
    assert n_edges % (_NW * _C) == 0
    return _make_edge_dot(n_edges)(zn, u, v)


# speedup vs baseline: 4.2589x; 3.0102x over previous
import functools

import jax
import jax.numpy as jnp
from jax import lax
from jax.experimental import pallas as pl
from jax.experimental.pallas import tpu as pltpu
from jax.experimental.pallas import tpu_sc as plsc

_NC = 2
_NS = 16
_NW = _NC * _NS
_L = 16
_D = 128
_C = 80


def _normalize_body(z_ref, out_ref):
    x = z_ref[...]
    ss = jnp.sum(x * x, axis=-1, keepdims=True)
    out_ref[...] = x / jnp.maximum(jnp.sqrt(ss), 1e-12)


def _make_edge_dot(n_edges: int):
    ew = n_edges // _NW
    n_chunks = ew // _C
    mesh = plsc.VectorSubcoreMesh(core_axis_name="c", subcore_axis_name="s")

    @functools.partial(
        pl.kernel,
        mesh=mesh,
        out_type=jax.ShapeDtypeStruct((n_edges,), jnp.float32),
        compiler_params=pltpu.CompilerParams(needs_layout_passes=False),
        scratch_types=[
            [
                dict(
                    idx_u=pltpu.VMEM((_C,), jnp.int32),
                    idx_v=pltpu.VMEM((_C,), jnp.int32),
                    rows_u=pltpu.VMEM((_C, _D), jnp.float32),
                    rows_v=pltpu.VMEM((_C, _D), jnp.float32),
                    oc=pltpu.VMEM((_C,), jnp.float32),
                    sem=pltpu.SemaphoreType.DMA,
                )
                for _ in range(2)
            ],
        ],
    )
    def edge_dot(zn, u, v, out, bufs):
        wid = lax.axis_index("s") * _NC + lax.axis_index("c")
        w0 = wid * ew

        def issue(t, b):
            base = pl.multiple_of(w0 + t * _C, 8)
            pltpu.sync_copy(u.at[pl.ds(base, _C)], b["idx_u"])
            pltpu.sync_copy(v.at[pl.ds(base, _C)], b["idx_v"])
            pltpu.async_copy(zn.at[b["idx_u"]], b["rows_u"], b["sem"])
            pltpu.async_copy(zn.at[b["idx_v"]], b["rows_v"], b["sem"])

        def drain(b):
            pltpu.make_async_copy(zn.at[b["idx_u"]], b["rows_u"], b["sem"]).wait()
            pltpu.make_async_copy(zn.at[b["idx_v"]], b["rows_v"], b["sem"]).wait()

        def compute(t, b):
            ru, rv, oc = b["rows_u"], b["rows_v"], b["oc"]

            lanes = lax.iota(jnp.int32, _L)

            def group_body(g, carry):
                r0 = pl.multiple_of(g * _L, _L)
                res = jnp.zeros((_L,), jnp.float32)
                for j in range(_L):
                    e = r0 + j
                    acc = ru[e, pl.ds(0, _L)] * rv[e, pl.ds(0, _L)]
                    for k in range(1, _D // _L):
                        acc = acc + ru[e, pl.ds(k * _L, _L)] * rv[e, pl.ds(k * _L, _L)]
                    res = jnp.where(lanes == j, jnp.sum(acc), res)
                oc[pl.ds(r0, _L)] = res
                return carry

            lax.fori_loop(0, _C // _L, group_body, 0)
            base = pl.multiple_of(w0 + t * _C, 8)
            pltpu.sync_copy(oc, out.at[pl.ds(base, _C)])

        ba, bb = bufs[0], bufs[1]
        issue(0, ba)

        def pipe_body(k, carry):
            t0 = k * 2
            issue(t0 + 1, bb)
            drain(ba)
            compute(t0, ba)
            issue(t0 + 2, ba)
            drain(bb)
            compute(t0 + 1, bb)
            return carry

        lax.fori_loop(0, (n_chunks - 1) // 2, pipe_body, 0)
        drain(ba)
        compute(n_chunks - 1, ba)

    return edge_dot


def kernel(z, edge_index):
    n, d = z.shape
    assert d == _D
    zn = pl.pallas_call(
        _normalize_body,
        out_shape=jax.ShapeDtypeStruct((n, d), jnp.float32),
    )(z)
    u = edge_index[0].astype(jnp.int32)
    v = edge_index[1].astype(jnp.int32)
    n_edges = u.shape[0]
    assert n_edges % (_NW * _C) == 0
    return _make_edge_dot(n_edges)(zn, u, v)


# speedup vs baseline: 5.1836x; 1.2171x over previous
import functools

import jax
import jax.numpy as jnp
from jax import lax
from jax.experimental import pallas as pl
from jax.experimental.pallas import tpu as pltpu
from jax.experimental.pallas import tpu_sc as plsc

_NC = 2
_NS = 16
_NW = _NC * _NS
_L = 16
_D = 128
_C = 80


def _normalize_body(z_ref, out_ref):
    x = z_ref[...]
    ss = jnp.sum(x * x, axis=-1, keepdims=True)
    out_ref[...] = x / jnp.maximum(jnp.sqrt(ss), 1e-12)


def _make_edge_dot(n_edges: int):
    ew = n_edges // _NW
    n_chunks = ew // _C
    mesh = plsc.VectorSubcoreMesh(core_axis_name="c", subcore_axis_name="s")

    @functools.partial(
        pl.kernel,
        mesh=mesh,
        out_type=jax.ShapeDtypeStruct((_NW, n_chunks, _C), jnp.float32),
        compiler_params=pltpu.CompilerParams(needs_layout_passes=False),
        scratch_types=[
            pltpu.VMEM((n_chunks, _C), jnp.int32),
            pltpu.VMEM((n_chunks, _C), jnp.int32),
            pltpu.VMEM((n_chunks, _C), jnp.float32),
            [
                dict(
                    rows_u=pltpu.VMEM((_C, _D), jnp.float32),
                    rows_v=pltpu.VMEM((_C, _D), jnp.float32),
                    sem=pltpu.SemaphoreType.DMA,
                )
                for _ in range(2)
            ],
        ],
    )
    def edge_dot(zn, u, v, out, idx_u, idx_v, oc, bufs):
        wid = lax.axis_index("s") * _NC + lax.axis_index("c")
        pltpu.sync_copy(u.at[wid], idx_u)
        pltpu.sync_copy(v.at[wid], idx_v)

        def issue(t, b):
            pltpu.async_copy(zn.at[idx_u.at[t]], b["rows_u"], b["sem"])
            pltpu.async_copy(zn.at[idx_v.at[t]], b["rows_v"], b["sem"])

        def drain(t, b):
            pltpu.make_async_copy(zn.at[idx_u.at[t]], b["rows_u"], b["sem"]).wait()
            pltpu.make_async_copy(zn.at[idx_v.at[t]], b["rows_v"], b["sem"]).wait()

        lanes = lax.iota(jnp.int32, _L)

        def compute(t, b):
            ru, rv = b["rows_u"], b["rows_v"]

            def group_body(g, carry):
                r0 = pl.multiple_of(g * _L, _L)
                res = jnp.zeros((_L,), jnp.float32)
                for j in range(_L):
                    e = r0 + j
                    acc = ru[e, pl.ds(0, _L)] * rv[e, pl.ds(0, _L)]
                    for k in range(1, _D // _L):
                        acc = acc + ru[e, pl.ds(k * _L, _L)] * rv[e, pl.ds(k * _L, _L)]
                    res = jnp.where(lanes == j, jnp.sum(acc), res)
                oc[t, pl.ds(r0, _L)] = res
                return carry

            lax.fori_loop(0, _C // _L, group_body, 0)

        ba, bb = bufs[0], bufs[1]
        issue(0, ba)

        def pipe_body(k, carry):
            t0 = k * 2
            issue(t0 + 1, bb)
            drain(t0, ba)
            compute(t0, ba)
            issue(t0 + 2, ba)
            drain(t0 + 1, bb)
            compute(t0 + 1, bb)
            return carry

        lax.fori_loop(0, (n_chunks - 1) // 2, pipe_body, 0)
        drain(n_chunks - 1, ba)
        compute(n_chunks - 1, ba)
        pltpu.sync_copy(oc, out.at[wid])

    return edge_dot


def kernel(z, edge_index):
    n, d = z.shape
    assert d == _D
    zn = pl.pallas_call(
        _normalize_body,
        out_shape=jax.ShapeDtypeStruct((n, d), jnp.float32),
    )(z)
    u = edge_index[0].astype(jnp.int32)
    v = edge_index[1].astype(jnp.int32)
    n_edges = u.shape[0]
    assert n_edges % (_NW * _C) == 0
    n_chunks = n_edges // (_NW * _C)
    u3 = u.reshape(_NW, n_chunks, _C)
    v3 = v.reshape(_NW, n_chunks, _C)
    out = _make_edge_dot(n_edges)(zn, u3, v3)
    return out.reshape(n_edges)
